# initial kernel scaffold (unmeasured)
import jax
import jax.numpy as jnp
from jax import lax
from jax.experimental import pallas as pl
from jax.experimental.pallas import tpu as pltpu

N_DEV = 4
N_TOK = 2048
D_MODEL = 512
H = 1024
N_EXP = 32
E_LOCAL = N_EXP // N_DEV
CAP = 51
CAP_PAD = 64
S = E_LOCAL * CAP_PAD
TOK_PER = N_TOK // N_DEV


def kernel(x, router_W, route_idx, expert_W):
    my = lax.axis_index("i")

    r = route_idx[:, 0]
    onehot = (r[:, None] == jnp.arange(N_EXP, dtype=jnp.int32)[None, :]).astype(jnp.int32)
    rank = jnp.sum(jnp.cumsum(onehot, axis=0) * onehot, axis=1) - 1
    col = jnp.where(rank < CAP, rank, CAP_PAD)
    ids_full = jnp.full((N_EXP, CAP_PAD), N_TOK, dtype=jnp.int32)
    ids_full = ids_full.at[r, col].set(jnp.arange(N_TOK, dtype=jnp.int32), mode="drop")
    ids = lax.dynamic_slice(ids_full, (my * E_LOCAL, 0), (E_LOCAL, CAP_PAD))
    ids = ids.reshape(S, 1)

    x_bf = x.astype(jnp.bfloat16)
    w_bf = expert_W.astype(jnp.bfloat16)

    def body(ids_ref, x_ref, w_ref, out_ref, send_buf, recv_buf, send_sems, recv_sems):
        me = lax.axis_index("i")

        barrier = pltpu.get_barrier_semaphore()
        for o in range(1, N_DEV):
            pl.semaphore_signal(
                barrier, inc=1,
                device_id=((me + o) % N_DEV,),
                device_id_type=pl.DeviceIdType.MESH,
            )
        pl.semaphore_wait(barrier, N_DEV - 1)

        ids_v = ids_ref[:, :]

        tok_iota = lax.broadcasted_iota(jnp.int32, (S, N_TOK), 1)
        d_full = (ids_v == tok_iota).astype(jnp.bfloat16)
        xg = lax.dot_general(
            d_full, x_ref[:, :], (((1,), (0,)), ((), ())),
            preferred_element_type=jnp.float32,
        ).astype(jnp.bfloat16)

        ygs = []
        for el in range(E_LOCAL):
            ygs.append(
                lax.dot_general(
                    xg[el * CAP_PAD:(el + 1) * CAP_PAD, :], w_ref[el],
                    (((1,), (0,)), ((), ())),
                    preferred_element_type=jnp.float32,
                ).astype(jnp.bfloat16)
            )
        yg = jnp.concatenate(ygs, axis=0)

        slab_iota = lax.broadcasted_iota(jnp.int32, (S, TOK_PER), 1)

        def slab_partial(j):
            dj = (ids_v == slab_iota + j * TOK_PER).astype(jnp.bfloat16)
            return lax.dot_general(
                dj, yg, (((0,), (0,)), ((), ())),
                preferred_element_type=jnp.float32,
            )

        sends = []
        for o in range(1, N_DEV):
            j = (me + o) % N_DEV
            send_buf[o - 1, :, :] = slab_partial(j).astype(jnp.bfloat16)
            rdma = pltpu.make_async_remote_copy(
                src_ref=send_buf.at[o - 1],
                dst_ref=recv_buf.at[me],
                send_sem=send_sems.at[o - 1],
                recv_sem=recv_sems.at[me],
                device_id=(j,),
                device_id_type=pl.DeviceIdType.MESH,
            )
            rdma.start()
            sends.append(rdma)

        out_ref[:, :] = slab_partial(me)

        for rdma in sends:
            rdma.wait_send()
        for s in range(N_DEV):
            @pl.when(s != me)
            def _():
                recv = pltpu.make_async_remote_copy(
                    src_ref=send_buf.at[0],
                    dst_ref=recv_buf.at[s],
                    send_sem=send_sems.at[0],
                    recv_sem=recv_sems.at[s],
                    device_id=(me,),
                    device_id_type=pl.DeviceIdType.MESH,
                )
                recv.wait_recv()
                out_ref[:, :] += recv_buf[s].astype(jnp.float32)

    return pl.pallas_call(
        body,
        out_shape=jax.ShapeDtypeStruct((TOK_PER, H), jnp.float32),
        in_specs=[
            pl.BlockSpec(memory_space=pltpu.VMEM),
            pl.BlockSpec(memory_space=pltpu.VMEM),
            pl.BlockSpec(memory_space=pltpu.VMEM),
        ],
        out_specs=pl.BlockSpec(memory_space=pltpu.VMEM),
        scratch_shapes=[
            pltpu.VMEM((N_DEV - 1, TOK_PER, H), jnp.bfloat16),
            pltpu.VMEM((N_DEV, TOK_PER, H), jnp.bfloat16),
            pltpu.SemaphoreType.DMA((N_DEV - 1,)),
            pltpu.SemaphoreType.DMA((N_DEV,)),
        ],
        compiler_params=pltpu.CompilerParams(collective_id=0),
    )(ids, x_bf, w_bf)


# baseline (device time: 55224 ns/iter reference)
import jax
import jax.numpy as jnp
from jax import lax
from jax.experimental import pallas as pl
from jax.experimental.pallas import tpu as pltpu

N_DEV = 4
N_TOK = 2048
D_MODEL = 512
H = 1024
N_EXP = 32
E_LOCAL = N_EXP // N_DEV
CAP = 51
CAP_PAD = 64
S = E_LOCAL * CAP_PAD
TOK_PER = N_TOK // N_DEV


def kernel(x, router_W, route_idx, expert_W):
    r = route_idx[:, 0]
    onehot = (r[:, None] == jnp.arange(N_EXP, dtype=jnp.int32)[None, :]).astype(jnp.int32)
    rank = jnp.sum(jnp.cumsum(onehot, axis=0) * onehot, axis=1) - 1
    rr = jnp.stack([r, rank], axis=0)

    x_bf = x.astype(jnp.bfloat16)
    w_bf = expert_W.astype(jnp.bfloat16)

    def body(rr_ref, x_ref, w_ref, out_ref, send_buf, recv_buf, send_sems, recv_sems):
        me = lax.axis_index("i")

        barrier = pltpu.get_barrier_semaphore()
        for o in range(1, N_DEV):
            pl.semaphore_signal(
                barrier, inc=1,
                device_id=((me + o) % N_DEV,),
                device_id_type=pl.DeviceIdType.MESH,
            )
        pl.semaphore_wait(barrier, N_DEV - 1)

        r_row = rr_ref[0:1, :]
        k_row = rr_ref[1:2, :]

        def dispatch(r_cols, k_cols, width):
            s_iota = lax.broadcasted_iota(jnp.int32, (S, width), 0)
            exp_s = (s_iota // CAP_PAD) + me * E_LOCAL
            cap_s = s_iota % CAP_PAD
            hit = (r_cols == exp_s) & (k_cols == cap_s) & (k_cols < CAP)
            return hit.astype(jnp.bfloat16)

        d_full = dispatch(r_row, k_row, N_TOK)
        xg = lax.dot_general(
            d_full, x_ref[:, :], (((1,), (0,)), ((), ())),
            preferred_element_type=jnp.float32,
        ).astype(jnp.bfloat16)

        ygs = []
        for el in range(E_LOCAL):
            ygs.append(
                lax.dot_general(
                    xg[el * CAP_PAD:(el + 1) * CAP_PAD, :], w_ref[el],
                    (((1,), (0,)), ((), ())),
                    preferred_element_type=jnp.float32,
                ).astype(jnp.bfloat16)
            )
        yg = jnp.concatenate(ygs, axis=0)

        def slab_partial(j):
            r_slab = rr_ref[0:1, pl.ds(j * TOK_PER, TOK_PER)]
            k_slab = rr_ref[1:2, pl.ds(j * TOK_PER, TOK_PER)]
            dj = dispatch(r_slab, k_slab, TOK_PER)
            return lax.dot_general(
                dj, yg, (((0,), (0,)), ((), ())),
                preferred_element_type=jnp.float32,
            )

        sends = []
        for o in range(1, N_DEV):
            j = (me + o) % N_DEV
            send_buf[o - 1, :, :] = slab_partial(j).astype(jnp.bfloat16)
            rdma = pltpu.make_async_remote_copy(
                src_ref=send_buf.at[o - 1],
                dst_ref=recv_buf.at[me],
                send_sem=send_sems.at[o - 1],
                recv_sem=recv_sems.at[me],
                device_id=(j,),
                device_id_type=pl.DeviceIdType.MESH,
            )
            rdma.start()
            sends.append(rdma)

        out_ref[:, :] = slab_partial(me)

        for rdma in sends:
            rdma.wait_send()
        for s in range(N_DEV):
            @pl.when(s != me)
            def _():
                recv = pltpu.make_async_remote_copy(
                    src_ref=send_buf.at[0],
                    dst_ref=recv_buf.at[s],
                    send_sem=send_sems.at[0],
                    recv_sem=recv_sems.at[s],
                    device_id=(me,),
                    device_id_type=pl.DeviceIdType.MESH,
                )
                recv.wait_recv()
                out_ref[:, :] += recv_buf[s].astype(jnp.float32)

    return pl.pallas_call(
        body,
        out_shape=jax.ShapeDtypeStruct((TOK_PER, H), jnp.float32),
        in_specs=[
            pl.BlockSpec(memory_space=pltpu.VMEM),
            pl.BlockSpec(memory_space=pltpu.VMEM),
            pl.BlockSpec(memory_space=pltpu.VMEM),
        ],
        out_specs=pl.BlockSpec(memory_space=pltpu.VMEM),
        scratch_shapes=[
            pltpu.VMEM((N_DEV - 1, TOK_PER, H), jnp.bfloat16),
            pltpu.VMEM((N_DEV, TOK_PER, H), jnp.bfloat16),
            pltpu.SemaphoreType.DMA((N_DEV - 1,)),
            pltpu.SemaphoreType.DMA((N_DEV,)),
        ],
        compiler_params=pltpu.CompilerParams(collective_id=0),
    )(rr, x_bf, w_bf)


# device time: 45379 ns/iter; 1.2170x vs baseline; 1.2170x over previous
import jax
import jax.numpy as jnp
from jax import lax
from jax.experimental import pallas as pl
from jax.experimental.pallas import tpu as pltpu

N_DEV = 4
N_TOK = 2048
D_MODEL = 512
H = 1024
N_EXP = 32
E_LOCAL = N_EXP // N_DEV
CAP = 51
CAP_PAD = 64
S = E_LOCAL * CAP_PAD
TOK_PER = N_TOK // N_DEV
RANK_CHUNK = 512


def kernel(x, router_W, route_idx, expert_W):
    r_row = route_idx.reshape(1, N_TOK)

    def body(rr_ref, rc_ref, x_ref, w_ref, out_ref,
             yg_ref, rank_ref, recv_buf, send_sems, recv_sems):
        me = lax.axis_index("i")

        barrier = pltpu.get_barrier_semaphore()
        for o in range(1, N_DEV):
            pl.semaphore_signal(
                barrier, inc=1,
                device_id=((me + o) % N_DEV,),
                device_id_type=pl.DeviceIdType.MESH,
            )
        pl.semaphore_wait(barrier, N_DEV - 1)

        rc = rc_ref[:, 0:1]
        for c0 in range(0, N_TOK, RANK_CHUNK):
            rr_c = rr_ref[0:1, c0:c0 + RANK_CHUNK]
            tp = lax.broadcasted_iota(jnp.int32, (N_TOK, RANK_CHUNK), 0)
            t = lax.broadcasted_iota(jnp.int32, (N_TOK, RANK_CHUNK), 1) + c0
            m = ((rc == rr_c) & (tp < t)).astype(jnp.int32)
            rank_ref[0:1, c0:c0 + RANK_CHUNK] = jnp.sum(m, axis=0, keepdims=True)

        def dispatch(r_cols, k_cols, width, eoff):
            s_iota = lax.broadcasted_iota(jnp.int32, (S, width), 0)
            exp_s = (s_iota // CAP_PAD) + eoff
            cap_s = s_iota % CAP_PAD
            hit = (r_cols == exp_s) & (k_cols == cap_s) & (k_cols < CAP)
            return hit.astype(jnp.bfloat16)

        x_bf = x_ref[:, :].astype(jnp.bfloat16)
        d_full = dispatch(rr_ref[0:1, :], rank_ref[0:1, :], N_TOK, me * E_LOCAL)
        xg = lax.dot_general(
            d_full, x_bf, (((1,), (0,)), ((), ())),
            preferred_element_type=jnp.float32,
        ).astype(jnp.bfloat16)

        ygs = []
        for el in range(E_LOCAL):
            y = lax.dot_general(
                xg[el * CAP_PAD:(el + 1) * CAP_PAD, :],
                w_ref[el].astype(jnp.bfloat16),
                (((1,), (0,)), ((), ())),
                preferred_element_type=jnp.float32,
            ).astype(jnp.bfloat16)
            ygs.append(y)
            yg_ref[el * CAP_PAD:(el + 1) * CAP_PAD, :] = y
        yg = jnp.concatenate(ygs, axis=0)

        sends = []
        for o in range(1, N_DEV):
            rdma = pltpu.make_async_remote_copy(
                src_ref=yg_ref,
                dst_ref=recv_buf.at[me],
                send_sem=send_sems.at[o - 1],
                recv_sem=recv_sems.at[me],
                device_id=((me + o) % N_DEV,),
                device_id_type=pl.DeviceIdType.MESH,
            )
            rdma.start()
            sends.append(rdma)

        r_slab = rr_ref[0:1, pl.ds(me * TOK_PER, TOK_PER)]
        k_slab = rank_ref[0:1, pl.ds(me * TOK_PER, TOK_PER)]
        d_own = dispatch(r_slab, k_slab, TOK_PER, me * E_LOCAL)
        out_ref[:, :] = lax.dot_general(
            d_own, yg, (((0,), (0,)), ((), ())),
            preferred_element_type=jnp.float32,
        )

        for rdma in sends:
            rdma.wait_send()
        for s in range(N_DEV):
            @pl.when(s != me)
            def _():
                recv = pltpu.make_async_remote_copy(
                    src_ref=yg_ref,
                    dst_ref=recv_buf.at[s],
                    send_sem=send_sems.at[0],
                    recv_sem=recv_sems.at[s],
                    device_id=(me,),
                    device_id_type=pl.DeviceIdType.MESH,
                )
                recv.wait_recv()
                d_s = dispatch(r_slab, k_slab, TOK_PER, s * E_LOCAL)
                out_ref[:, :] += lax.dot_general(
                    d_s, recv_buf[s], (((0,), (0,)), ((), ())),
                    preferred_element_type=jnp.float32,
                )

    return pl.pallas_call(
        body,
        out_shape=jax.ShapeDtypeStruct((TOK_PER, H), jnp.float32),
        in_specs=[
            pl.BlockSpec(memory_space=pltpu.VMEM),
            pl.BlockSpec(memory_space=pltpu.VMEM),
            pl.BlockSpec(memory_space=pltpu.VMEM),
            pl.BlockSpec(memory_space=pltpu.VMEM),
        ],
        out_specs=pl.BlockSpec(memory_space=pltpu.VMEM),
        scratch_shapes=[
            pltpu.VMEM((S, H), jnp.bfloat16),
            pltpu.VMEM((1, N_TOK), jnp.int32),
            pltpu.VMEM((N_DEV, S, H), jnp.bfloat16),
            pltpu.SemaphoreType.DMA((N_DEV - 1,)),
            pltpu.SemaphoreType.DMA((N_DEV,)),
        ],
        compiler_params=pltpu.CompilerParams(collective_id=0),
    )(r_row, route_idx, x, expert_W)


# device time: 40700 ns/iter; 1.3569x vs baseline; 1.1150x over previous
import jax
import jax.numpy as jnp
from jax import lax
from jax.experimental import pallas as pl
from jax.experimental.pallas import tpu as pltpu

N_DEV = 4
N_TOK = 2048
D_MODEL = 512
H = 1024
N_EXP = 32
E_LOCAL = N_EXP // N_DEV
CAP = 51
CAP_PAD = 52
S = E_LOCAL * CAP_PAD
TOK_PER = N_TOK // N_DEV
RANK_CHUNK = 512


def kernel(x, router_W, route_idx, expert_W):
    r_row = route_idx.reshape(1, N_TOK)

    def body(rr_ref, rc_ref, x_ref, w_ref, out_ref,
             yg_ref, rank_ref, recv_buf, send_sems, recv_sems):
        me = lax.axis_index("i")

        barrier = pltpu.get_barrier_semaphore()
        for o in range(1, N_DEV):
            pl.semaphore_signal(
                barrier, inc=1,
                device_id=((me + o) % N_DEV,),
                device_id_type=pl.DeviceIdType.MESH,
            )
        pl.semaphore_wait(barrier, N_DEV - 1)

        rc = rc_ref[:, 0:1]
        ones_row = jnp.ones((1, N_TOK), jnp.bfloat16)
        tp = lax.broadcasted_iota(jnp.int32, (N_TOK, RANK_CHUNK), 0)
        tl = lax.broadcasted_iota(jnp.int32, (N_TOK, RANK_CHUNK), 1)
        tdiff = tp - tl
        for c0 in range(0, N_TOK, RANK_CHUNK):
            rr_c = rr_ref[0:1, c0:c0 + RANK_CHUNK]
            m = ((rc == rr_c) & (tdiff < c0)).astype(jnp.bfloat16)
            rank_ref[0:1, c0:c0 + RANK_CHUNK] = lax.dot_general(
                ones_row, m, (((1,), (0,)), ((), ())),
                preferred_element_type=jnp.float32,
            ).astype(jnp.int32)

        s_iota_w = lax.broadcasted_iota(jnp.int32, (S, N_TOK), 0)
        exp_w = s_iota_w // CAP_PAD
        cap_w = s_iota_w % CAP_PAD
        s_iota_n = lax.broadcasted_iota(jnp.int32, (S, TOK_PER), 0)
        exp_n = s_iota_n // CAP_PAD
        cap_n = s_iota_n % CAP_PAD

        x_bf = x_ref[:, :].astype(jnp.bfloat16)
        k_full = rank_ref[0:1, :]
        d_full = (
            (rr_ref[0:1, :] == exp_w + me * E_LOCAL)
            & (k_full == cap_w) & (k_full < CAP)
        ).astype(jnp.bfloat16)
        xg = lax.dot_general(
            d_full, x_bf, (((1,), (0,)), ((), ())),
            preferred_element_type=jnp.float32,
        ).astype(jnp.bfloat16)

        ygs = []
        for el in range(E_LOCAL):
            y = lax.dot_general(
                xg[el * CAP_PAD:(el + 1) * CAP_PAD, :],
                w_ref[el].astype(jnp.bfloat16),
                (((1,), (0,)), ((), ())),
                preferred_element_type=jnp.float32,
            ).astype(jnp.bfloat16)
            ygs.append(y)
            yg_ref[el * CAP_PAD:(el + 1) * CAP_PAD, :] = y
        yg = jnp.concatenate(ygs, axis=0)

        sends = []
        for o in range(1, N_DEV):
            rdma = pltpu.make_async_remote_copy(
                src_ref=yg_ref,
                dst_ref=recv_buf.at[me],
                send_sem=send_sems.at[o - 1],
                recv_sem=recv_sems.at[me],
                device_id=((me + o) % N_DEV,),
                device_id_type=pl.DeviceIdType.MESH,
            )
            rdma.start()
            sends.append(rdma)

        r_slab = rr_ref[0:1, pl.ds(me * TOK_PER, TOK_PER)]
        k_slab = rank_ref[0:1, pl.ds(me * TOK_PER, TOK_PER)]
        k_hit = (k_slab == cap_n) & (k_slab < CAP)

        def dispatch_slab(eoff):
            return ((r_slab == exp_n + eoff) & k_hit).astype(jnp.bfloat16)

        d_by_src = [dispatch_slab(s * E_LOCAL) for s in range(N_DEV)]
        d_own = dispatch_slab(me * E_LOCAL)
        out_ref[:, :] = lax.dot_general(
            d_own, yg, (((0,), (0,)), ((), ())),
            preferred_element_type=jnp.float32,
        )

        for rdma in sends:
            rdma.wait_send()
        for s in range(N_DEV):
            @pl.when(s != me)
            def _():
                recv = pltpu.make_async_remote_copy(
                    src_ref=yg_ref,
                    dst_ref=recv_buf.at[s],
                    send_sem=send_sems.at[0],
                    recv_sem=recv_sems.at[s],
                    device_id=(me,),
                    device_id_type=pl.DeviceIdType.MESH,
                )
                recv.wait_recv()
                out_ref[:, :] += lax.dot_general(
                    d_by_src[s], recv_buf[s], (((0,), (0,)), ((), ())),
                    preferred_element_type=jnp.float32,
                )

    return pl.pallas_call(
        body,
        out_shape=jax.ShapeDtypeStruct((TOK_PER, H), jnp.float32),
        in_specs=[
            pl.BlockSpec(memory_space=pltpu.VMEM),
            pl.BlockSpec(memory_space=pltpu.VMEM),
            pl.BlockSpec(memory_space=pltpu.VMEM),
            pl.BlockSpec(memory_space=pltpu.VMEM),
        ],
        out_specs=pl.BlockSpec(memory_space=pltpu.VMEM),
        scratch_shapes=[
            pltpu.VMEM((S, H), jnp.bfloat16),
            pltpu.VMEM((1, N_TOK), jnp.int32),
            pltpu.VMEM((N_DEV, S, H), jnp.bfloat16),
            pltpu.SemaphoreType.DMA((N_DEV - 1,)),
            pltpu.SemaphoreType.DMA((N_DEV,)),
        ],
        compiler_params=pltpu.CompilerParams(collective_id=0),
    )(r_row, route_idx, x, expert_W)


# device time: 38925 ns/iter; 1.4187x vs baseline; 1.0456x over previous
import jax
import jax.numpy as jnp
from jax import lax
from jax.experimental import pallas as pl
from jax.experimental.pallas import tpu as pltpu

N_DEV = 4
N_TOK = 2048
D_MODEL = 512
H = 1024
N_EXP = 32
E_LOCAL = N_EXP // N_DEV
CAP = 51
CAP_PAD = 52
S = E_LOCAL * CAP_PAD
TOK_PER = N_TOK // N_DEV
RC = 512
N_CHUNK = E_LOCAL // 2


def kernel(x, router_W, route_idx, expert_W):
    r_row = route_idx.reshape(1, N_TOK)

    def body(rr_ref, rc_ref, x_ref, w_ref, out_ref,
             yg_ref, rank_ref, recv_buf, send_sems, recv_sems):
        me = lax.axis_index("i")

        barrier = pltpu.get_barrier_semaphore()
        for o in range(1, N_DEV):
            pl.semaphore_signal(
                barrier, inc=1,
                device_id=((me + o) % N_DEV,),
                device_id_type=pl.DeviceIdType.MESH,
            )
        pl.semaphore_wait(barrier, N_DEV - 1)

        tri = (
            lax.broadcasted_iota(jnp.int32, (RC, RC), 0)
            < lax.broadcasted_iota(jnp.int32, (RC, RC), 1)
        )
        e_col = lax.broadcasted_iota(jnp.int32, (N_EXP, RC), 0)
        cum = jnp.zeros((N_EXP, 1), jnp.float32)
        for c0 in range(0, N_TOK, RC):
            rr_c = rr_ref[0:1, c0:c0 + RC]
            rc_c = rc_ref[c0:c0 + RC, 0:1]
            m = ((rc_c == rr_c) & tri).astype(jnp.float32)
            intra = jnp.sum(m, axis=0, keepdims=True)
            oh_t = (e_col == rr_c).astype(jnp.float32)
            offs = jnp.sum(oh_t * cum, axis=0, keepdims=True)
            rank_ref[0:1, c0:c0 + RC] = (intra + offs).astype(jnp.int32)
            cum = cum + jnp.sum(oh_t, axis=1, keepdims=True)

        s_iota_w = lax.broadcasted_iota(jnp.int32, (S, N_TOK), 0)
        exp_w = s_iota_w // CAP_PAD
        cap_w = s_iota_w % CAP_PAD
        s_iota_n = lax.broadcasted_iota(jnp.int32, (S, TOK_PER), 0)
        exp_n = s_iota_n // CAP_PAD
        cap_n = s_iota_n % CAP_PAD

        x_bf = x_ref[:, :].astype(jnp.bfloat16)
        k_full = rank_ref[0:1, :]
        d_full = (
            (rr_ref[0:1, :] == exp_w + me * E_LOCAL)
            & (k_full == cap_w) & (k_full < CAP)
        ).astype(jnp.bfloat16)
        xg = lax.dot_general(
            d_full, x_bf, (((1,), (0,)), ((), ())),
            preferred_element_type=jnp.float32,
        ).astype(jnp.bfloat16)

        sends = []
        ygs = []
        for el in range(E_LOCAL):
            y = lax.dot_general(
                xg[el * CAP_PAD:(el + 1) * CAP_PAD, :],
                w_ref[el].astype(jnp.bfloat16),
                (((1,), (0,)), ((), ())),
                preferred_element_type=jnp.float32,
            ).astype(jnp.bfloat16)
            ygs.append(y)
            yg_ref[el * CAP_PAD:(el + 1) * CAP_PAD, :] = y
            if el % 2 == 1:
                ch = el // 2
                for o in range(1, N_DEV):
                    rdma = pltpu.make_async_remote_copy(
                        src_ref=yg_ref.at[pl.ds(ch * 2 * CAP_PAD, 2 * CAP_PAD)],
                        dst_ref=recv_buf.at[pl.ds(me * S + ch * 2 * CAP_PAD, 2 * CAP_PAD)],
                        send_sem=send_sems.at[(o - 1) * N_CHUNK + ch],
                        recv_sem=recv_sems.at[me * N_CHUNK + ch],
                        device_id=((me + o) % N_DEV,),
                        device_id_type=pl.DeviceIdType.MESH,
                    )
                    rdma.start()
                    sends.append(rdma)
        yg = jnp.concatenate(ygs, axis=0)

        r_slab = rr_ref[0:1, pl.ds(me * TOK_PER, TOK_PER)]
        k_slab = rank_ref[0:1, pl.ds(me * TOK_PER, TOK_PER)]
        k_hit = (k_slab == cap_n) & (k_slab < CAP)
        d_by_src = [
            ((r_slab == exp_n + s * E_LOCAL) & k_hit).astype(jnp.bfloat16)
            for s in range(N_DEV)
        ]

        for s in range(N_DEV):
            @pl.when(s == me)
            def _():
                out_ref[:, :] = lax.dot_general(
                    d_by_src[s], yg, (((0,), (0,)), ((), ())),
                    preferred_element_type=jnp.float32,
                )

        for rdma in sends:
            rdma.wait_send()
        for s in range(N_DEV):
            @pl.when(s != me)
            def _():
                for ch in range(N_CHUNK):
                    recv = pltpu.make_async_remote_copy(
                        src_ref=yg_ref.at[pl.ds(ch * 2 * CAP_PAD, 2 * CAP_PAD)],
                        dst_ref=recv_buf.at[pl.ds(s * S + ch * 2 * CAP_PAD, 2 * CAP_PAD)],
                        send_sem=send_sems.at[0],
                        recv_sem=recv_sems.at[s * N_CHUNK + ch],
                        device_id=(me,),
                        device_id_type=pl.DeviceIdType.MESH,
                    )
                    recv.wait_recv()
                out_ref[:, :] += lax.dot_general(
                    d_by_src[s], recv_buf[s * S:(s + 1) * S, :],
                    (((0,), (0,)), ((), ())),
                    preferred_element_type=jnp.float32,
                )

    return pl.pallas_call(
        body,
        out_shape=jax.ShapeDtypeStruct((TOK_PER, H), jnp.float32),
        in_specs=[
            pl.BlockSpec(memory_space=pltpu.VMEM),
            pl.BlockSpec(memory_space=pltpu.VMEM),
            pl.BlockSpec(memory_space=pltpu.VMEM),
            pl.BlockSpec(memory_space=pltpu.VMEM),
        ],
        out_specs=pl.BlockSpec(memory_space=pltpu.VMEM),
        scratch_shapes=[
            pltpu.VMEM((S, H), jnp.bfloat16),
            pltpu.VMEM((1, N_TOK), jnp.int32),
            pltpu.VMEM((N_DEV * S, H), jnp.bfloat16),
            pltpu.SemaphoreType.DMA(((N_DEV - 1) * N_CHUNK,)),
            pltpu.SemaphoreType.DMA((N_DEV * N_CHUNK,)),
        ],
        compiler_params=pltpu.CompilerParams(collective_id=0),
    )(r_row, route_idx, x, expert_W)


# device time: 32466 ns/iter; 1.7010x vs baseline; 1.1989x over previous
import jax
import jax.numpy as jnp
from jax import lax
from jax.experimental import pallas as pl
from jax.experimental.pallas import tpu as pltpu

N_DEV = 4
N_TOK = 2048
D_MODEL = 512
H = 1024
N_EXP = 32
E_LOCAL = N_EXP // N_DEV
CAP = 51
CAP_PAD = 52
S = E_LOCAL * CAP_PAD
TOK_PER = N_TOK // N_DEV
RC = 512
N_CHUNK = E_LOCAL // 2


def kernel(x, router_W, route_idx, expert_W):
    r_row = route_idx.reshape(1, N_TOK)

    def body(rr_ref, rc_ref, x_hbm, w_hbm, out_ref,
             yg_ref, rank_ref, recv_buf, x_buf, w_buf,
             copy_sems, send_sems, recv_sems):
        me = lax.axis_index("i")

        cp_x = pltpu.make_async_copy(x_hbm, x_buf, copy_sems.at[E_LOCAL])
        cp_x.start()
        cp_w = []
        for el in range(E_LOCAL):
            cp = pltpu.make_async_copy(w_hbm.at[el], w_buf.at[el], copy_sems.at[el])
            cp.start()
            cp_w.append(cp)

        barrier = pltpu.get_barrier_semaphore()
        for o in range(1, N_DEV):
            pl.semaphore_signal(
                barrier, inc=1,
                device_id=((me + o) % N_DEV,),
                device_id_type=pl.DeviceIdType.MESH,
            )

        tri = (
            lax.broadcasted_iota(jnp.int32, (RC, RC), 0)
            < lax.broadcasted_iota(jnp.int32, (RC, RC), 1)
        )
        e_col = lax.broadcasted_iota(jnp.int32, (N_EXP, RC), 0)
        cum = jnp.zeros((N_EXP, 1), jnp.float32)
        for c0 in range(0, N_TOK, RC):
            rr_c = rr_ref[0:1, c0:c0 + RC]
            rc_c = rc_ref[c0:c0 + RC, 0:1]
            m = ((rc_c == rr_c) & tri).astype(jnp.float32)
            intra = jnp.sum(m, axis=0, keepdims=True)
            oh_t = (e_col == rr_c).astype(jnp.float32)
            offs = jnp.sum(oh_t * cum, axis=0, keepdims=True)
            rank_ref[0:1, c0:c0 + RC] = (intra + offs).astype(jnp.int32)
            cum = cum + jnp.sum(oh_t, axis=1, keepdims=True)

        s_iota_w = lax.broadcasted_iota(jnp.int32, (S, N_TOK), 0)
        exp_w = s_iota_w // CAP_PAD
        cap_w = s_iota_w % CAP_PAD
        s_iota_n = lax.broadcasted_iota(jnp.int32, (S, TOK_PER), 0)
        exp_n = s_iota_n // CAP_PAD
        cap_n = s_iota_n % CAP_PAD

        cp_x.wait()
        x_bf = x_buf[:, :].astype(jnp.bfloat16)
        k_full = rank_ref[0:1, :]
        d_full = (
            (rr_ref[0:1, :] == exp_w + me * E_LOCAL)
            & (k_full == cap_w) & (k_full < CAP)
        ).astype(jnp.bfloat16)
        xg = lax.dot_general(
            d_full, x_bf, (((1,), (0,)), ((), ())),
            preferred_element_type=jnp.float32,
        ).astype(jnp.bfloat16)

        pl.semaphore_wait(barrier, N_DEV - 1)

        sends = []
        ygs = []
        for el in range(E_LOCAL):
            cp_w[el].wait()
            y = lax.dot_general(
                xg[el * CAP_PAD:(el + 1) * CAP_PAD, :],
                w_buf[el].astype(jnp.bfloat16),
                (((1,), (0,)), ((), ())),
                preferred_element_type=jnp.float32,
            ).astype(jnp.bfloat16)
            ygs.append(y)
            yg_ref[el * CAP_PAD:(el + 1) * CAP_PAD, :] = y
            if el % 2 == 1:
                ch = el // 2
                for o in range(1, N_DEV):
                    rdma = pltpu.make_async_remote_copy(
                        src_ref=yg_ref.at[pl.ds(ch * 2 * CAP_PAD, 2 * CAP_PAD)],
                        dst_ref=recv_buf.at[pl.ds(me * S + ch * 2 * CAP_PAD, 2 * CAP_PAD)],
                        send_sem=send_sems.at[(o - 1) * N_CHUNK + ch],
                        recv_sem=recv_sems.at[me * N_CHUNK + ch],
                        device_id=((me + o) % N_DEV,),
                        device_id_type=pl.DeviceIdType.MESH,
                    )
                    rdma.start()
                    sends.append(rdma)
        yg = jnp.concatenate(ygs, axis=0)

        r_slab = rr_ref[0:1, pl.ds(me * TOK_PER, TOK_PER)]
        k_slab = rank_ref[0:1, pl.ds(me * TOK_PER, TOK_PER)]
        k_hit = (k_slab == cap_n) & (k_slab < CAP)
        d_by_src = [
            ((r_slab == exp_n + s * E_LOCAL) & k_hit).astype(jnp.bfloat16)
            for s in range(N_DEV)
        ]

        for s in range(N_DEV):
            @pl.when(s == me)
            def _():
                out_ref[:, :] = lax.dot_general(
                    d_by_src[s], yg, (((0,), (0,)), ((), ())),
                    preferred_element_type=jnp.float32,
                ).astype(jnp.bfloat16)

        for rdma in sends:
            rdma.wait_send()
        for s in range(N_DEV):
            @pl.when(s != me)
            def _():
                for ch in range(N_CHUNK):
                    recv = pltpu.make_async_remote_copy(
                        src_ref=yg_ref.at[pl.ds(ch * 2 * CAP_PAD, 2 * CAP_PAD)],
                        dst_ref=recv_buf.at[pl.ds(s * S + ch * 2 * CAP_PAD, 2 * CAP_PAD)],
                        send_sem=send_sems.at[0],
                        recv_sem=recv_sems.at[s * N_CHUNK + ch],
                        device_id=(me,),
                        device_id_type=pl.DeviceIdType.MESH,
                    )
                    recv.wait_recv()
                acc = lax.dot_general(
                    d_by_src[s], recv_buf[s * S:(s + 1) * S, :],
                    (((0,), (0,)), ((), ())),
                    preferred_element_type=jnp.float32,
                )
                out_ref[:, :] = (out_ref[:, :] + acc.astype(jnp.bfloat16))

    return pl.pallas_call(
        body,
        out_shape=jax.ShapeDtypeStruct((TOK_PER, H), jnp.bfloat16),
        in_specs=[
            pl.BlockSpec(memory_space=pltpu.VMEM),
            pl.BlockSpec(memory_space=pltpu.VMEM),
            pl.BlockSpec(memory_space=pl.ANY),
            pl.BlockSpec(memory_space=pl.ANY),
        ],
        out_specs=pl.BlockSpec(memory_space=pltpu.VMEM),
        scratch_shapes=[
            pltpu.VMEM((S, H), jnp.bfloat16),
            pltpu.VMEM((1, N_TOK), jnp.int32),
            pltpu.VMEM((N_DEV * S, H), jnp.bfloat16),
            pltpu.VMEM((N_TOK, D_MODEL), jnp.float32),
            pltpu.VMEM((E_LOCAL, D_MODEL, H), jnp.float32),
            pltpu.SemaphoreType.DMA((E_LOCAL + 1,)),
            pltpu.SemaphoreType.DMA(((N_DEV - 1) * N_CHUNK,)),
            pltpu.SemaphoreType.DMA((N_DEV * N_CHUNK,)),
        ],
        compiler_params=pltpu.CompilerParams(collective_id=0),
    )(r_row, route_idx, x, expert_W)
